# Initial kernel scaffold; baseline (speedup 1.0000x reference)
import functools
import math

import jax
import jax.numpy as jnp
from jax import lax
from jax.experimental import pallas as pl
from jax.experimental.pallas import tpu as pltpu
from jax.experimental.pallas import tpu_sc as plsc

_D = 128
_LANES = 16
_SCALE = math.sqrt(float(_D))
_NC = 2
_NS = 16
_NW = _NC * _NS
_CHUNK = 128
_NBUF = 5


@functools.partial(jax.jit, static_argnums=(2,))
def _embed(xr, table, n_chunks):
    rows_per_w = n_chunks * _CHUNK
    total = _NW * rows_per_w
    n_groups = n_chunks // _NBUF
    mesh = plsc.VectorSubcoreMesh(
        core_axis_name="c", subcore_axis_name="s",
        num_cores=_NC, num_subcores=_NS)

    @functools.partial(
        pl.kernel,
        mesh=mesh,
        out_type=jax.ShapeDtypeStruct((total, _D), jnp.float32),
        scratch_types=(
            [pltpu.VMEM((n_chunks, _CHUNK), jnp.int32),
             pltpu.VMEM((_NBUF, _CHUNK, _D), jnp.float32)]
            + [pltpu.SemaphoreType.DMA] * (2 * _NBUF)
        ),
    )
    def body(x_hbm, tab_hbm, out_hbm, idx_v, bufs, *sems):
        gsems = sems[:_NBUF]
        osems = sems[_NBUF:]
        wid = lax.axis_index("s") * _NC + lax.axis_index("c")
        base = wid * rows_per_w

        pltpu.sync_copy(x_hbm.at[wid], idx_v)

        def start_gather(j, b):
            pltpu.async_copy(tab_hbm.at[idx_v.at[j]], bufs.at[b], gsems[b])

        def wait_gather(b):
            pltpu.make_async_copy(
                tab_hbm.at[pl.ds(0, _CHUNK)], bufs.at[b], gsems[b]).wait()

        def start_out(j, b):
            pltpu.async_copy(
                bufs.at[b], out_hbm.at[pl.ds(base + j * _CHUNK, _CHUNK)],
                osems[b])

        def wait_out(b):
            pltpu.make_async_copy(
                bufs.at[b], out_hbm.at[pl.ds(0, _CHUNK)], osems[b]).wait()

        def scale(b):
            buf = bufs.at[b]

            def row(i, c):
                for k in range(_D // _LANES):
                    sl = pl.ds(k * _LANES, _LANES)
                    buf[i, sl] = buf[i, sl] * _SCALE
                return c

            lax.fori_loop(0, _CHUNK, row, 0)

        for b in range(_NBUF):
            start_gather(b, b)

        def group(p, carry):
            for b in range(_NBUF):
                j = p * _NBUF + b
                wait_gather(b)
                scale(b)

                @pl.when(p > 0)
                def _():
                    wait_out(b)

                start_out(j, b)

                @pl.when(p < n_groups - 1)
                def _():
                    start_gather(j + _NBUF, b)

            return carry

        lax.fori_loop(0, n_groups, group, 0)

        for b in range(_NBUF):
            wait_out(b)

    return body(xr, table)


def kernel(x, table):
    b, s = x.shape
    xf = x.reshape(-1).astype(jnp.int32)
    total = xf.shape[0]
    quantum = _NW * _CHUNK * _NBUF
    padded = -(-total // quantum) * quantum
    if padded != total:
        xf = jnp.concatenate(
            [xf, jnp.zeros((padded - total,), jnp.int32)])
    n_chunks = padded // (_NW * _CHUNK)
    xr = xf.reshape(_NW, n_chunks, _CHUNK)
    out = _embed(xr, table, n_chunks)
    return out[:total].reshape(b, s, _D)


# speedup vs baseline: 2.4185x; 2.4185x over previous
import functools
import math

import jax
import jax.numpy as jnp
from jax import lax
from jax.experimental import pallas as pl
from jax.experimental.pallas import tpu as pltpu
from jax.experimental.pallas import tpu_sc as plsc

_D = 128
_LANES = 16
_SCALE = math.sqrt(float(_D))
_NC = 2
_NS = 16
_NW = _NC * _NS
_CHUNK = 128
_NBUF = 5


@functools.partial(jax.jit, static_argnums=(2,))
def _embed(xr, table, n_chunks):
    rows_per_w = n_chunks * _CHUNK
    total = _NW * rows_per_w
    n_groups = n_chunks // _NBUF
    mesh = plsc.VectorSubcoreMesh(
        core_axis_name="c", subcore_axis_name="s",
        num_cores=_NC, num_subcores=_NS)

    @functools.partial(
        pl.kernel,
        mesh=mesh,
        out_type=jax.ShapeDtypeStruct((total, _D), jnp.float32),
        scratch_types=(
            [pltpu.VMEM((n_chunks, _CHUNK), jnp.int32),
             pltpu.VMEM((_NBUF, _CHUNK, _D), jnp.float32)]
            + [pltpu.SemaphoreType.DMA] * (2 * _NBUF)
        ),
    )
    def body(x_hbm, tab_hbm, out_hbm, idx_v, bufs, *sems):
        gsems = sems[:_NBUF]
        osems = sems[_NBUF:]
        wid = lax.axis_index("s") * _NC + lax.axis_index("c")
        base = wid * rows_per_w

        pltpu.sync_copy(x_hbm.at[wid], idx_v)

        def start_gather(j, b):
            pltpu.async_copy(tab_hbm.at[idx_v.at[j]], bufs.at[b], gsems[b])

        def wait_gather(b):
            pltpu.make_async_copy(
                tab_hbm.at[pl.ds(0, _CHUNK)], bufs.at[b], gsems[b]).wait()

        def start_out(j, b):
            pltpu.async_copy(
                bufs.at[b], out_hbm.at[pl.ds(base + j * _CHUNK, _CHUNK)],
                osems[b])

        def wait_out(b):
            pltpu.make_async_copy(
                bufs.at[b], out_hbm.at[pl.ds(0, _CHUNK)], osems[b]).wait()

        def scale(b):
            buf = bufs.at[b]

            def row(i, c):
                for k in range(_D // _LANES):
                    sl = pl.ds(k * _LANES, _LANES)
                    buf[i, sl] = buf[i, sl] * _SCALE
                return c

            lax.fori_loop(0, _CHUNK, row, 0)

        def chunk(j, carry):
            pltpu.async_copy(
                tab_hbm.at[idx_v.at[j]], bufs.at[0], gsems[0]).wait()
            scale(0)
            pltpu.sync_copy(
                bufs.at[0], out_hbm.at[pl.ds(base + j * _CHUNK, _CHUNK)])
            return carry

        lax.fori_loop(0, n_chunks, chunk, 0)

    return body(xr, table)


def kernel(x, table):
    b, s = x.shape
    xf = x.reshape(-1).astype(jnp.int32)
    total = xf.shape[0]
    quantum = _NW * _CHUNK * _NBUF
    padded = -(-total // quantum) * quantum
    if padded != total:
        xf = jnp.concatenate(
            [xf, jnp.zeros((padded - total,), jnp.int32)])
    n_chunks = padded // (_NW * _CHUNK)
    xr = xf.reshape(_NW, n_chunks, _CHUNK)
    out = _embed(xr, table, n_chunks)
    return out[:total].reshape(b, s, _D)


# speedup vs baseline: 2.9649x; 1.2259x over previous
import functools
import math

import jax
import jax.numpy as jnp
from jax import lax
from jax.experimental import pallas as pl
from jax.experimental.pallas import tpu as pltpu
from jax.experimental.pallas import tpu_sc as plsc

_D = 128
_LANES = 16
_SCALE = math.sqrt(float(_D))
_NC = 2
_NS = 16
_NW = _NC * _NS
_CHUNK = 128
_NBUF = 5


@functools.partial(jax.jit, static_argnums=(2,))
def _embed(xr, table, n_chunks):
    rows_per_w = n_chunks * _CHUNK
    total = _NW * rows_per_w
    n_groups = n_chunks // _NBUF
    mesh = plsc.VectorSubcoreMesh(
        core_axis_name="c", subcore_axis_name="s",
        num_cores=_NC, num_subcores=_NS)

    @functools.partial(
        pl.kernel,
        mesh=mesh,
        out_type=jax.ShapeDtypeStruct((total, _D), jnp.float32),
        scratch_types=(
            [pltpu.VMEM((n_chunks, _CHUNK), jnp.int32),
             pltpu.VMEM((_NBUF, _CHUNK, _D), jnp.float32)]
            + [pltpu.SemaphoreType.DMA] * (2 * _NBUF)
        ),
    )
    def body(x_hbm, tab_hbm, out_hbm, idx_v, bufs, *sems):
        gsems = sems[:_NBUF]
        osems = sems[_NBUF:]
        wid = lax.axis_index("s") * _NC + lax.axis_index("c")
        base = wid * rows_per_w

        pltpu.sync_copy(x_hbm.at[wid], idx_v)

        def start_gather(j, b):
            pltpu.async_copy(tab_hbm.at[idx_v.at[j]], bufs.at[b], gsems[b])

        def wait_gather(b):
            pltpu.make_async_copy(
                tab_hbm.at[idx_v.at[0]], bufs.at[b], gsems[b]).wait()

        def start_out(j, b):
            pltpu.async_copy(
                bufs.at[b], out_hbm.at[pl.ds(base + j * _CHUNK, _CHUNK)],
                osems[b])

        def wait_out(b):
            pltpu.make_async_copy(
                bufs.at[b], out_hbm.at[pl.ds(0, _CHUNK)], osems[b]).wait()

        def scale(b):
            buf = bufs.at[b]

            def row(i, c):
                for k in range(_D // _LANES):
                    sl = pl.ds(k * _LANES, _LANES)
                    buf[i, sl] = buf[i, sl] * _SCALE
                return c

            lax.fori_loop(0, _CHUNK, row, 0)

        start_gather(0, 0)
        start_gather(1, 1)

        def group(p, carry):
            for b in range(_NBUF):
                j = p * _NBUF + b
                b2 = (b + 2) % _NBUF

                @pl.when(j >= 3)
                def _():
                    wait_out(b2)

                @pl.when(j + 2 < n_chunks)
                def _():
                    start_gather(j + 2, b2)

                wait_gather(b)
                scale(b)
                start_out(j, b)

            return carry

        lax.fori_loop(0, n_groups, group, 0)

        for t in range(3):
            wait_out((n_chunks - 3 + t) % _NBUF)

    return body(xr, table)


def kernel(x, table):
    b, s = x.shape
    xf = x.reshape(-1).astype(jnp.int32)
    total = xf.shape[0]
    quantum = _NW * _CHUNK * _NBUF
    padded = -(-total // quantum) * quantum
    if padded != total:
        xf = jnp.concatenate(
            [xf, jnp.zeros((padded - total,), jnp.int32)])
    n_chunks = padded // (_NW * _CHUNK)
    xr = xf.reshape(_NW, n_chunks, _CHUNK)
    out = _embed(xr, table, n_chunks)
    return out[:total].reshape(b, s, _D)


# speedup vs baseline: 5.1046x; 1.7217x over previous
import functools
import math

import jax
import jax.numpy as jnp
from jax import lax
from jax.experimental import pallas as pl
from jax.experimental.pallas import tpu as pltpu
from jax.experimental.pallas import tpu_sc as plsc

_D = 128
_LANES = 16
_SCALE = math.sqrt(float(_D))
_NC = 2
_NS = 16
_NW = _NC * _NS
_NBUF = 4


@jax.jit
def _embed(x, table):
    nb, seq = x.shape
    bpw = nb // _NW
    n_chunks = bpw
    mesh = plsc.VectorSubcoreMesh(
        core_axis_name="c", subcore_axis_name="s",
        num_cores=_NC, num_subcores=_NS)

    @functools.partial(
        pl.kernel,
        mesh=mesh,
        out_type=jax.ShapeDtypeStruct((nb, seq, _D), jnp.float32),
        scratch_types=(
            [pltpu.VMEM((bpw, seq), jnp.int32),
             pltpu.VMEM((_NBUF, seq, _D), jnp.float32)]
            + [pltpu.SemaphoreType.DMA] * (2 * _NBUF)
        ),
    )
    def body(x_hbm, tab_hbm, out_hbm, idx_v, bufs, *sems):
        gsems = sems[:_NBUF]
        osems = sems[_NBUF:]
        wid = lax.axis_index("s") * _NC + lax.axis_index("c")
        base = wid * bpw

        pltpu.sync_copy(x_hbm.at[pl.ds(base, bpw)], idx_v)

        def start_gather(j, b):
            pltpu.async_copy(tab_hbm.at[idx_v.at[j]], bufs.at[b], gsems[b])

        def wait_gather(b):
            pltpu.make_async_copy(
                tab_hbm.at[idx_v.at[0]], bufs.at[b], gsems[b]).wait()

        def start_out(j, b):
            pltpu.async_copy(bufs.at[b], out_hbm.at[base + j], osems[b])

        def wait_out(b):
            pltpu.make_async_copy(
                bufs.at[b], out_hbm.at[0], osems[b]).wait()

        def scale(b):
            buf = bufs.at[b]

            def row(i, c):
                for k in range(_D // _LANES):
                    sl = pl.ds(k * _LANES, _LANES)
                    buf[i, sl] = buf[i, sl] * _SCALE
                return c

            lax.fori_loop(0, seq, row, 0)

        start_gather(0, 0)
        start_gather(1, 1)

        def group(p, carry):
            for b in range(_NBUF):
                j = p * _NBUF + b
                b2 = (b + 2) % _NBUF

                @pl.when(j >= 2)
                def _():
                    wait_out(b2)

                @pl.when(j + 2 < n_chunks)
                def _():
                    start_gather(j + 2, b2)

                wait_gather(b)
                scale(b)
                start_out(j, b)

            return carry

        lax.fori_loop(0, n_chunks // _NBUF, group, 0)

        for t in range(2):
            wait_out((n_chunks - 2 + t) % _NBUF)

    return body(x, table)


def kernel(x, table):
    b, s = x.shape
    x = x.astype(jnp.int32)
    quantum = _NW * _NBUF
    padded = -(-b // quantum) * quantum
    if padded != b:
        x = jnp.concatenate([x, jnp.zeros((padded - b, s), jnp.int32)])
    out = _embed(x, table)
    return out[:b] if padded != b else out


# speedup vs baseline: 5.1191x; 1.0028x over previous
import functools
import math

import jax
import jax.numpy as jnp
from jax import lax
from jax.experimental import pallas as pl
from jax.experimental.pallas import tpu as pltpu
from jax.experimental.pallas import tpu_sc as plsc

_D = 128
_LANES = 16
_SCALE = math.sqrt(float(_D))
_NC = 2
_NS = 16
_NW = _NC * _NS
_NBUF = 4


@jax.jit
def _embed(x, table):
    nb, seq = x.shape
    bpw = nb // _NW
    n_chunks = bpw
    mesh = plsc.VectorSubcoreMesh(
        core_axis_name="c", subcore_axis_name="s",
        num_cores=_NC, num_subcores=_NS)

    @functools.partial(
        pl.kernel,
        mesh=mesh,
        out_type=jax.ShapeDtypeStruct((nb, seq, _D), jnp.float32),
        compiler_params=pltpu.CompilerParams(use_tc_tiling_on_sc=True),
        scratch_types=(
            [pltpu.VMEM((bpw, seq), jnp.int32),
             pltpu.VMEM((_NBUF, seq, _D), jnp.float32)]
            + [pltpu.SemaphoreType.DMA] * (2 * _NBUF)
        ),
    )
    def body(x_hbm, tab_hbm, out_hbm, idx_v, bufs, *sems):
        gsems = sems[:_NBUF]
        osems = sems[_NBUF:]
        wid = lax.axis_index("s") * _NC + lax.axis_index("c")
        base = wid * bpw

        pltpu.sync_copy(x_hbm.at[pl.ds(base, bpw)], idx_v)

        def start_gather(j, b):
            pltpu.async_copy(tab_hbm.at[idx_v.at[j]], bufs.at[b], gsems[b])

        def wait_gather(b):
            pltpu.make_async_copy(
                tab_hbm.at[idx_v.at[0]], bufs.at[b], gsems[b]).wait()

        def start_out(j, b):
            pltpu.async_copy(bufs.at[b], out_hbm.at[base + j], osems[b])

        def wait_out(b):
            pltpu.make_async_copy(
                bufs.at[b], out_hbm.at[0], osems[b]).wait()

        def scale(b):
            buf = bufs.at[b]

            def row(i, c):
                for k in range(_D // _LANES):
                    sl = pl.ds(k * _LANES, _LANES)
                    buf[i, sl] = buf[i, sl] * _SCALE
                return c

            lax.fori_loop(0, seq, row, 0)

        start_gather(0, 0)
        start_gather(1, 1)

        def group(p, carry):
            for b in range(_NBUF):
                j = p * _NBUF + b
                b2 = (b + 2) % _NBUF

                @pl.when(j >= 2)
                def _():
                    wait_out(b2)

                @pl.when(j + 2 < n_chunks)
                def _():
                    start_gather(j + 2, b2)

                wait_gather(b)
                scale(b)
                start_out(j, b)

            return carry

        lax.fori_loop(0, n_chunks // _NBUF, group, 0)

        for t in range(2):
            wait_out((n_chunks - 2 + t) % _NBUF)

    return body(x, table)


def kernel(x, table):
    b, s = x.shape
    x = x.astype(jnp.int32)
    quantum = _NW * _NBUF
    padded = -(-b // quantum) * quantum
    if padded != b:
        x = jnp.concatenate([x, jnp.zeros((padded - b, s), jnp.int32)])
    out = _embed(x, table)
    return out[:b] if padded != b else out


# speedup vs baseline: 9.4449x; 1.8451x over previous
import functools
import math

import jax
import jax.numpy as jnp
from jax import lax
from jax.experimental import pallas as pl
from jax.experimental.pallas import tpu as pltpu
from jax.experimental.pallas import tpu_sc as plsc

_D = 128
_LANES = 16
_SCALE = math.sqrt(float(_D))
_NC = 2
_NS = 16
_NW = _NC * _NS
_NBUF = 6
_SKEW = 3


@jax.jit
def _embed(xt, table):
    seq, nb = xt.shape
    bpw = nb // _NW
    head = min(8, seq)
    mesh = plsc.VectorSubcoreMesh(
        core_axis_name="c", subcore_axis_name="s",
        num_cores=_NC, num_subcores=_NS)

    @functools.partial(
        pl.kernel,
        mesh=mesh,
        out_type=jax.ShapeDtypeStruct((seq, nb, _D), jnp.float32),
        scratch_types=(
            [pltpu.VMEM((seq, bpw), jnp.int32),
             pltpu.VMEM((_NBUF, bpw, _D), jnp.float32)]
            + [pltpu.SemaphoreType.DMA] * (2 * _NBUF)
        ),
    )
    def body(xt_hbm, tab_hbm, out_hbm, idx_v, bufs, *sems):
        gsems = sems[:_NBUF]
        osems = sems[_NBUF:]
        wid = lax.axis_index("s") * _NC + lax.axis_index("c")
        col0 = wid * bpw

        def start_gather(j, b):
            pltpu.async_copy(tab_hbm.at[idx_v.at[j]], bufs.at[b], gsems[b])

        def wait_gather(b):
            pltpu.make_async_copy(
                tab_hbm.at[idx_v.at[0]], bufs.at[b], gsems[b]).wait()

        def start_out(j, b):
            pltpu.async_copy(
                bufs.at[b], out_hbm.at[j, pl.ds(col0, bpw)], osems[b])

        def wait_out(b):
            pltpu.make_async_copy(
                bufs.at[b], out_hbm.at[0, pl.ds(col0, bpw)], osems[b]).wait()

        def scale(b):
            buf = bufs.at[b]

            def row(i, c):
                for k in range(_D // _LANES):
                    sl = pl.ds(k * _LANES, _LANES)
                    buf[i, sl] = buf[i, sl] * _SCALE
                return c

            lax.fori_loop(0, bpw, row, 0)

        pltpu.sync_copy(xt_hbm.at[:, pl.ds(col0, bpw)], idx_v)
        for j in range(min(_SKEW, seq)):
            start_gather(j, j)

        def visit(j, b, static):
            b2 = (b + _SKEW) % _NBUF
            if static:
                if j >= _SKEW:
                    wait_out(b2)
                if j + _SKEW < seq:
                    start_gather(j + _SKEW, b2)
            else:
                @pl.when(j >= _SKEW)
                def _():
                    wait_out(b2)

                @pl.when(j + _SKEW < seq)
                def _():
                    start_gather(j + _SKEW, b2)

            wait_gather(b)
            scale(b)
            start_out(j, b)

        n_groups = seq // _NBUF

        def group(p, carry):
            for b in range(_NBUF):
                visit(p * _NBUF + b, b, False)
            return carry

        lax.fori_loop(0, n_groups, group, 0)

        for j in range(n_groups * _NBUF, seq):
            visit(j, j % _NBUF, True)

        for t in range(min(_SKEW, seq)):
            wait_out((seq - min(_SKEW, seq) + t) % _NBUF)

    return body(xt, table)


def kernel(x, table):
    b, s = x.shape
    x = x.astype(jnp.int32)
    padded = -(-b // _NW) * _NW
    if padded != b:
        x = jnp.concatenate([x, jnp.zeros((padded - b, s), jnp.int32)])
    out_t = _embed(jnp.swapaxes(x, 0, 1), table)
    out = jnp.swapaxes(out_t, 0, 1)
    return out[:b] if padded != b else out


# speedup vs baseline: 9.4450x; 1.0000x over previous
import functools
import math

import jax
import jax.numpy as jnp
from jax import lax
from jax.experimental import pallas as pl
from jax.experimental.pallas import tpu as pltpu
from jax.experimental.pallas import tpu_sc as plsc

_D = 128
_LANES = 16
_SCALE = math.sqrt(float(_D))
_NC = 2
_NS = 16
_NW = _NC * _NS
_NBUF = 6
_SKEW = 3


@jax.jit
def _embed(xt, table):
    seq, nb = xt.shape
    bpw = nb // _NW
    mesh = plsc.VectorSubcoreMesh(
        core_axis_name="c", subcore_axis_name="s",
        num_cores=_NC, num_subcores=_NS)

    @functools.partial(
        pl.kernel,
        mesh=mesh,
        out_type=jax.ShapeDtypeStruct((seq * nb, _D), jnp.float32),
        scratch_types=(
            [pltpu.VMEM((seq, bpw), jnp.int32),
             pltpu.VMEM((_NBUF, bpw), jnp.int32),
             pltpu.VMEM((_NBUF, bpw, _D), jnp.float32)]
            + [pltpu.SemaphoreType.DMA] * (2 * _NBUF)
        ),
    )
    def body(xt_hbm, tab_hbm, out_hbm, idx_v, oidx_v, bufs, *sems):
        gsems = sems[:_NBUF]
        osems = sems[_NBUF:]
        wid = lax.axis_index("s") * _NC + lax.axis_index("c")
        col0 = wid * bpw

        def start_gather(j, b):
            pltpu.async_copy(tab_hbm.at[idx_v.at[j]], bufs.at[b], gsems[b])

        def wait_gather(b):
            pltpu.make_async_copy(
                tab_hbm.at[idx_v.at[0]], bufs.at[b], gsems[b]).wait()

        def start_out(j, b):
            base = j * nb + col0
            for c in range(bpw // _LANES):
                oidx_v[b, pl.ds(c * _LANES, _LANES)] = (
                    lax.iota(jnp.int32, _LANES) + (base + c * _LANES))
            pltpu.async_copy(bufs.at[b], out_hbm.at[oidx_v.at[b]], osems[b])

        def wait_out(b):
            pltpu.make_async_copy(
                bufs.at[b], out_hbm.at[oidx_v.at[b]], osems[b]).wait()

        def scale(b):
            buf = bufs.at[b]

            def row(i, c):
                for k in range(_D // _LANES):
                    sl = pl.ds(k * _LANES, _LANES)
                    buf[i, sl] = buf[i, sl] * _SCALE
                return c

            lax.fori_loop(0, bpw, row, 0)

        pltpu.sync_copy(xt_hbm.at[:, pl.ds(col0, bpw)], idx_v)
        for j in range(min(_SKEW, seq)):
            start_gather(j, j)

        def visit(j, b, static):
            b2 = (b + _SKEW) % _NBUF
            if static:
                if j >= _SKEW:
                    wait_out(b2)
                if j + _SKEW < seq:
                    start_gather(j + _SKEW, b2)
            else:
                @pl.when(j >= _SKEW)
                def _():
                    wait_out(b2)

                @pl.when(j + _SKEW < seq)
                def _():
                    start_gather(j + _SKEW, b2)

            wait_gather(b)
            scale(b)
            start_out(j, b)

        n_groups = seq // _NBUF

        def group(p, carry):
            for b in range(_NBUF):
                visit(p * _NBUF + b, b, False)
            return carry

        lax.fori_loop(0, n_groups, group, 0)

        for j in range(n_groups * _NBUF, seq):
            visit(j, j % _NBUF, True)

        for t in range(min(_SKEW, seq)):
            wait_out((seq - min(_SKEW, seq) + t) % _NBUF)

    return body(xt, table)


def kernel(x, table):
    b, s = x.shape
    x = x.astype(jnp.int32)
    padded = -(-b // _NW) * _NW
    if padded != b:
        x = jnp.concatenate([x, jnp.zeros((padded - b, s), jnp.int32)])
    flat = _embed(jnp.swapaxes(x, 0, 1), table)
    out = jnp.swapaxes(flat.reshape(s, padded, _D), 0, 1)
    return out[:b] if padded != b else out
